# baseline (speedup 1.0000x reference)
import functools

import jax
import jax.numpy as jnp
from jax import lax
from jax.experimental import pallas as pl
from jax.experimental.pallas import tpu as pltpu
from jax.experimental.pallas import tpu_sc as plsc

L = 2048
D_IN = 128
D_OUT = 192
D_EDGE_IN = 16
D_EDGE_OUT = 32
NN = 16
D_HN = 256
D_HE = 128
K_EDGE = D_EDGE_OUT - D_EDGE_IN
K_NODE = D_OUT - D_IN

RB = 256
TB = RB * NN
GRID = L // RB

_EPS = 1e-5


def _fold_bn(p):
    a = p['scale'] * lax.rsqrt(p['var'] + _EPS)
    b = p['bias'] - p['mean'] * a
    return a.reshape(1, -1), b.reshape(1, -1)


def _rb_flat(p):
    a1, b1 = _fold_bn(p['bn1'])
    a2, b2 = _fold_bn(p['bn2'])
    out = [a1, b1, p['conv1']['w'], p['conv1']['b'].reshape(1, -1),
           a2, b2, p['conv2']['w'], p['conv2']['b'].reshape(1, -1)]
    if 'shortcut_conv' in p:
        asc, bsc = _fold_bn(p['shortcut_bn'])
        wsc = p['shortcut_conv']['w'] * asc.reshape(-1, 1)
        csc = bsc @ p['shortcut_conv']['w'] + p['shortcut_conv']['b'].reshape(1, -1)
        out += [wsc, csc]
    return out


def _mm(a, w):
    return jnp.dot(a.astype(jnp.bfloat16), w.astype(jnp.bfloat16),
                   preferred_element_type=jnp.float32)


def _resblock(h, a1, b1, w1, c1, a2, b2, w2, c2):
    t = jnp.maximum(h * a1 + b1, 0.0)
    t = _mm(t, w1) + c1
    t = jnp.maximum(t * a2 + b2, 0.0)
    t = _mm(t, w2) + c2
    return h + t


def _resblock_sc(h, a1, b1, w1, c1, a2, b2, w2, c2, wsc, csc):
    t = jnp.maximum(h * a1 + b1, 0.0)
    t = _mm(t, w1) + c1
    t = jnp.maximum(t * a2 + b2, 0.0)
    t = _mm(t, w2) + c2
    sc = _mm(h, wsc) + csc
    return t + sc


def _mlp_body(xb_ref, evf_ref, ntg_ref, *refs):
    w = [r[...] for r in refs[:-2]]
    out_ref, ev2_ref = refs[-2], refs[-1]

    xb = xb_ref[...]
    evf = evf_ref[...]
    ntg = ntg_ref[...]

    (we_self, we_ev, we_nt, be0,
     e_a1, e_b1, e_w1, e_c1, e_a2, e_b2, e_w2, e_c2,
     f_a1, f_b1, f_w1, f_c1, f_a2, f_b2, f_w2, f_c2, f_wsc, f_csc,
     ae, be,
     wn_self, wn_ev, wn_nt, bn0) = w[:28]
    rbn0 = w[28:36]
    rbn1 = w[36:44]
    an, bn = w[44:46]
    rbr0 = w[46:54]
    rbr1 = w[54:62]
    rbrf = w[62:72]
    ar, br = w[72:74]

    def rep(v):
        c = v.shape[1]
        return jnp.broadcast_to(v[:, None, :], (RB, NN, c)).reshape(TB, c)

    se = _mm(xb, we_self)
    h = (rep(se)
         + _mm(evf, we_ev)
         + _mm(ntg, we_nt)
         + be0)
    h = _resblock(h, e_a1, e_b1, e_w1, e_c1, e_a2, e_b2, e_w2, e_c2)
    h = _resblock_sc(h, f_a1, f_b1, f_w1, f_c1, f_a2, f_b2, f_w2, f_c2,
                     f_wsc, f_csc)
    nen = jnp.maximum(h * ae + be, 0.0)
    ev2 = jnp.concatenate([evf, nen], axis=1)
    ev2_ref[...] = ev2.reshape(RB, NN, D_EDGE_OUT)

    ns = _mm(xb, wn_self)
    h = (rep(ns)
         + _mm(ev2, wn_ev)
         + _mm(ntg, wn_nt)
         + bn0)
    h = _resblock(h, *rbn0)
    h = _resblock(h, *rbn1)
    h = jnp.maximum(h * an + bn, 0.0)
    agg = jnp.sum(h.reshape(RB, NN, D_HN), axis=1)

    r = _resblock(agg, *rbr0)
    r = _resblock(r, *rbr1)
    r = _resblock_sc(r, *rbrf)
    r = jnp.maximum(r * ar + br, 0.0)
    out_ref[...] = jnp.concatenate([xb, r], axis=1)


def _flatten_params(params):
    flat = [params['conv_e0']['w'][:D_IN],
            params['conv_e0']['w'][D_IN:D_IN + D_EDGE_IN],
            params['conv_e0']['w'][D_IN + D_EDGE_IN:],
            params['conv_e0']['b'].reshape(1, -1)]
    flat += _rb_flat(params['rb_e'][0])
    flat += _rb_flat(params['rb_e_final'])
    ae, be = _fold_bn(params['bn_e'])
    flat += [ae, be]
    flat += [params['conv_n0']['w'][:D_IN],
             params['conv_n0']['w'][D_IN:D_IN + D_EDGE_OUT],
             params['conv_n0']['w'][D_IN + D_EDGE_OUT:],
             params['conv_n0']['b'].reshape(1, -1)]
    flat += _rb_flat(params['rb_n'][0])
    flat += _rb_flat(params['rb_n'][1])
    an, bn = _fold_bn(params['bn_n'])
    flat += [an, bn]
    flat += _rb_flat(params['rb_r'][0])
    flat += _rb_flat(params['rb_r'][1])
    flat += _rb_flat(params['rb_r_final'])
    ar, br = _fold_bn(params['bn_r'])
    flat += [ar, br]
    return flat


def _full_spec(arr):
    nd = arr.ndim
    return pl.BlockSpec(arr.shape, lambda i, _n=nd: (0,) * _n)


def _mlp_body_alias(xb_ref, evf_ref, ntg_ref, *refs):
    _mlp_body(xb_ref, evf_ref, ntg_ref, *refs[2:])


def _mlp(x, evf, nodetrg, flat, h, prev):
    off = h * (_LH // RB)
    in_specs = [pl.BlockSpec((RB, D_IN), lambda i: (i + off, 0)),
                pl.BlockSpec((TB, D_EDGE_IN), lambda i: (i + off, 0)),
                pl.BlockSpec((TB, D_IN), lambda i: (i, 0))]
    extra = ()
    aliases = {}
    body = _mlp_body
    if prev is not None:
        in_specs += [
            pl.BlockSpec((RB, D_OUT), lambda i: (i + off, 0)),
            pl.BlockSpec((RB, NN, D_EDGE_OUT), lambda i: (i + off, 0, 0))]
        extra = (prev[0], prev[1])
        aliases = {3: 0, 4: 1}
        body = _mlp_body_alias
    in_specs += [_full_spec(a) for a in flat]
    out_specs = [pl.BlockSpec((RB, D_OUT), lambda i: (i + off, 0)),
                 pl.BlockSpec((RB, NN, D_EDGE_OUT), lambda i: (i + off, 0, 0))]
    out, ev2 = pl.pallas_call(
        body,
        grid=(_LH // RB,),
        in_specs=in_specs,
        out_specs=out_specs,
        out_shape=[jax.ShapeDtypeStruct((L, D_OUT), jnp.float32),
                   jax.ShapeDtypeStruct((L, NN, D_EDGE_OUT), jnp.float32)],
        input_output_aliases=aliases,
    )(x, evf, nodetrg, *extra, *flat)
    return out, ev2


_NW = 32
_NH = 2
_LH = L // _NH
_RPW = _LH // _NW
_IPW = _RPW * NN
_ABATCH = 8
_GCHUNK = 128
_WPR = L // 4


def _sc_body(adj_hbm, x_hbm, out_hbm, adjbuf0, adjbuf1, idxrow, idxall,
             rows0, rows1, sa0, sa1, sg0, sg1, so0, so1):
    wid = lax.axis_index("s") * 2 + lax.axis_index("c")
    row0 = wid * _RPW
    adjb = [adjbuf0, adjbuf1]
    rows = [rows0, rows1]
    sa, sg, so = [sa0, sa1], [sg0, sg1], [so0, so1]
    nb = _RPW // _ABATCH

    def start_adj(b):
        return pltpu.async_copy(
            adj_hbm.at[pl.ds((row0 + b * _ABATCH) * L, _ABATCH * L)],
            adjb[b % 2], sa[b % 2])

    def start_gather(b):
        return pltpu.async_copy(
            x_hbm.at[idxall.at[pl.ds(b * _GCHUNK, _GCHUNK)]],
            rows[b % 2], sg[b % 2])

    def start_out(b):
        return pltpu.async_copy(
            rows[b % 2],
            out_hbm.at[pl.ds(wid * _IPW + b * _GCHUNK, _GCHUNK)], so[b % 2])

    def scan_batch(buf, b):
        def scan_row(r, _):
            rbase = r * L

            def chunk(carry, want_true):
                k, ptr = carry
                for u in range(2):
                    vals = buf[pl.ds(rbase + (k + u) * NN, NN)]
                    mask = (vals != 0) if want_true else (vals == 0)
                    inds = lax.iota(jnp.int32, NN) + (k + u) * NN
                    plsc.store_compressed(idxrow.at[pl.ds(ptr, NN)], inds,
                                          mask=mask)
                    ptr = ptr + jnp.sum(mask.astype(jnp.int32))
                return k + 2, ptr

            def cont(carry):
                k, ptr = carry
                return jnp.logical_and(ptr < NN, k < L // NN)

            _, ptr = lax.while_loop(cont, lambda c: chunk(c, True), (0, 0))
            _, _ = lax.while_loop(cont, lambda c: chunk(c, False), (0, ptr))
            row = b * _ABATCH + r
            idxall[pl.ds(row * NN, NN)] = idxrow[pl.ds(0, NN)]
            return 0

        lax.fori_loop(0, _ABATCH, scan_row, 0)

    h_adj, h_g, h_o = {}, {}, {}
    h_adj[0] = start_adj(0)
    for b in range(nb):
        if b + 1 < nb:
            h_adj[b + 1] = start_adj(b + 1)
        h_adj[b].wait()
        scan_batch(adjb[b % 2], b)
        if b >= 1:
            h_g[b - 1].wait()
            h_o[b - 1] = start_out(b - 1)
        if b >= 2:
            h_o[b - 2].wait()
        h_g[b] = start_gather(b)
    h_g[nb - 1].wait()
    h_o[nb - 1] = start_out(nb - 1)
    h_o[nb - 2].wait()
    h_o[nb - 1].wait()


def _neighbors(x, adj_i32):
    mesh = plsc.VectorSubcoreMesh(core_axis_name="c", subcore_axis_name="s")
    f = functools.partial(
        pl.kernel,
        mesh=mesh,
        compiler_params=pltpu.CompilerParams(needs_layout_passes=False,
                                             use_tc_tiling_on_sc=False),
        out_type=jax.ShapeDtypeStruct((_LH * NN, D_IN // 2), jnp.int32),
        scratch_types=[
            pltpu.VMEM((_ABATCH * L,), jnp.int32),
            pltpu.VMEM((_ABATCH * L,), jnp.int32),
            pltpu.VMEM((3 * NN,), jnp.int32),
            pltpu.VMEM((_IPW,), jnp.int32),
            pltpu.VMEM((_GCHUNK, D_IN // 2), jnp.int32),
            pltpu.VMEM((_GCHUNK, D_IN // 2), jnp.int32),
            pltpu.SemaphoreType.DMA,
            pltpu.SemaphoreType.DMA,
            pltpu.SemaphoreType.DMA,
            pltpu.SemaphoreType.DMA,
            pltpu.SemaphoreType.DMA,
            pltpu.SemaphoreType.DMA,
        ],
    )(_sc_body)
    return f(adj_i32, x)


@jax.jit
def kernel(x, edgevec, adjmat, params):
    evf = edgevec.reshape(L * NN, D_EDGE_IN)
    xbf = lax.bitcast_convert_type(
        x.astype(jnp.bfloat16).reshape(L, D_IN // 2, 2), jnp.int32)
    flat = _flatten_params(params)
    prev = None
    for h in range(_NH):
        adj_h = adjmat[h * _LH:(h + 1) * _LH].reshape(_LH * L).astype(jnp.int32)
        ntg_h = lax.bitcast_convert_type(
            _neighbors(xbf, adj_h), jnp.bfloat16).reshape(_LH * NN, D_IN)
        prev = _mlp(x, evf, ntg_h, flat, h, prev)
    out, ev2 = prev
    return out, ev2


# speedup vs baseline: 1.6825x; 1.6825x over previous
import functools

import jax
import jax.numpy as jnp
from jax import lax
from jax.experimental import pallas as pl
from jax.experimental.pallas import tpu as pltpu
from jax.experimental.pallas import tpu_sc as plsc

L = 2048
D_IN = 128
D_OUT = 192
D_EDGE_IN = 16
D_EDGE_OUT = 32
NN = 16
D_HN = 256
D_HE = 128
K_EDGE = D_EDGE_OUT - D_EDGE_IN
K_NODE = D_OUT - D_IN

RB = 256
TB = RB * NN
GRID = L // RB

_EPS = 1e-5


def _fold_bn(p):
    a = p['scale'] * lax.rsqrt(p['var'] + _EPS)
    b = p['bias'] - p['mean'] * a
    return a.reshape(1, -1), b.reshape(1, -1)


def _rb_flat(p):
    a1, b1 = _fold_bn(p['bn1'])
    a2, b2 = _fold_bn(p['bn2'])
    out = [a1, b1, p['conv1']['w'], p['conv1']['b'].reshape(1, -1),
           a2, b2, p['conv2']['w'], p['conv2']['b'].reshape(1, -1)]
    if 'shortcut_conv' in p:
        asc, bsc = _fold_bn(p['shortcut_bn'])
        wsc = p['shortcut_conv']['w'] * asc.reshape(-1, 1)
        csc = bsc @ p['shortcut_conv']['w'] + p['shortcut_conv']['b'].reshape(1, -1)
        out += [wsc, csc]
    return out


def _mm(a, w):
    return jnp.dot(a.astype(jnp.bfloat16), w.astype(jnp.bfloat16),
                   preferred_element_type=jnp.float32)


def _resblock(h, a1, b1, w1, c1, a2, b2, w2, c2):
    t = jnp.maximum(h * a1 + b1, 0.0)
    t = _mm(t, w1) + c1
    t = jnp.maximum(t * a2 + b2, 0.0)
    t = _mm(t, w2) + c2
    return h + t


def _resblock_sc(h, a1, b1, w1, c1, a2, b2, w2, c2, wsc, csc):
    t = jnp.maximum(h * a1 + b1, 0.0)
    t = _mm(t, w1) + c1
    t = jnp.maximum(t * a2 + b2, 0.0)
    t = _mm(t, w2) + c2
    sc = _mm(h, wsc) + csc
    return t + sc


def _mlp_body(xb_ref, evf_ref, ntg_ref, *refs):
    w = [r[...] for r in refs[:-2]]
    out_ref, ev2_ref = refs[-2], refs[-1]

    xb = xb_ref[...]
    evf = evf_ref[...]
    ntg = ntg_ref[...]

    (we_self, we_ev, we_nt, be0,
     e_a1, e_b1, e_w1, e_c1, e_a2, e_b2, e_w2, e_c2,
     f_a1, f_b1, f_w1, f_c1, f_a2, f_b2, f_w2, f_c2, f_wsc, f_csc,
     ae, be,
     wn_self, wn_ev, wn_nt, bn0) = w[:28]
    rbn0 = w[28:36]
    rbn1 = w[36:44]
    an, bn = w[44:46]
    rbr0 = w[46:54]
    rbr1 = w[54:62]
    rbrf = w[62:72]
    ar, br = w[72:74]

    def rep(v):
        c = v.shape[1]
        return jnp.broadcast_to(v[:, None, :], (RB, NN, c)).reshape(TB, c)

    se = _mm(xb, we_self)
    h = (rep(se)
         + _mm(evf, we_ev)
         + _mm(ntg, we_nt)
         + be0)
    h = _resblock(h, e_a1, e_b1, e_w1, e_c1, e_a2, e_b2, e_w2, e_c2)
    h = _resblock_sc(h, f_a1, f_b1, f_w1, f_c1, f_a2, f_b2, f_w2, f_c2,
                     f_wsc, f_csc)
    nen = jnp.maximum(h * ae + be, 0.0)
    ev2 = jnp.concatenate([evf, nen], axis=1)
    ev2_ref[...] = ev2.reshape(RB, NN, D_EDGE_OUT)

    ns = _mm(xb, wn_self)
    h = (rep(ns)
         + _mm(ev2, wn_ev)
         + _mm(ntg, wn_nt)
         + bn0)
    h = _resblock(h, *rbn0)
    h = _resblock(h, *rbn1)
    h = jnp.maximum(h * an + bn, 0.0)
    agg = jnp.sum(h.reshape(RB, NN, D_HN), axis=1)

    r = _resblock(agg, *rbr0)
    r = _resblock(r, *rbr1)
    r = _resblock_sc(r, *rbrf)
    r = jnp.maximum(r * ar + br, 0.0)
    out_ref[...] = jnp.concatenate([xb, r], axis=1)


def _flatten_params(params):
    flat = [params['conv_e0']['w'][:D_IN],
            params['conv_e0']['w'][D_IN:D_IN + D_EDGE_IN],
            params['conv_e0']['w'][D_IN + D_EDGE_IN:],
            params['conv_e0']['b'].reshape(1, -1)]
    flat += _rb_flat(params['rb_e'][0])
    flat += _rb_flat(params['rb_e_final'])
    ae, be = _fold_bn(params['bn_e'])
    flat += [ae, be]
    flat += [params['conv_n0']['w'][:D_IN],
             params['conv_n0']['w'][D_IN:D_IN + D_EDGE_OUT],
             params['conv_n0']['w'][D_IN + D_EDGE_OUT:],
             params['conv_n0']['b'].reshape(1, -1)]
    flat += _rb_flat(params['rb_n'][0])
    flat += _rb_flat(params['rb_n'][1])
    an, bn = _fold_bn(params['bn_n'])
    flat += [an, bn]
    flat += _rb_flat(params['rb_r'][0])
    flat += _rb_flat(params['rb_r'][1])
    flat += _rb_flat(params['rb_r_final'])
    ar, br = _fold_bn(params['bn_r'])
    flat += [ar, br]
    return flat


def _full_spec(arr):
    nd = arr.ndim
    return pl.BlockSpec(arr.shape, lambda i, _n=nd: (0,) * _n)


def _mlp_body_alias(xb_ref, evf_ref, ntg_ref, *refs):
    _mlp_body(xb_ref, evf_ref, ntg_ref, *refs[2:])


def _mlp(x, evf, nodetrg, flat, h, prev):
    off = h * (_LH // RB)
    in_specs = [pl.BlockSpec((RB, D_IN), lambda i: (i + off, 0)),
                pl.BlockSpec((TB, D_EDGE_IN), lambda i: (i + off, 0)),
                pl.BlockSpec((TB, D_IN), lambda i: (i, 0))]
    extra = ()
    aliases = {}
    body = _mlp_body
    if prev is not None:
        in_specs += [
            pl.BlockSpec((RB, D_OUT), lambda i: (i + off, 0)),
            pl.BlockSpec((RB, NN, D_EDGE_OUT), lambda i: (i + off, 0, 0))]
        extra = (prev[0], prev[1])
        aliases = {3: 0, 4: 1}
        body = _mlp_body_alias
    in_specs += [_full_spec(a) for a in flat]
    out_specs = [pl.BlockSpec((RB, D_OUT), lambda i: (i + off, 0)),
                 pl.BlockSpec((RB, NN, D_EDGE_OUT), lambda i: (i + off, 0, 0))]
    out, ev2 = pl.pallas_call(
        body,
        grid=(_LH // RB,),
        in_specs=in_specs,
        out_specs=out_specs,
        out_shape=[jax.ShapeDtypeStruct((L, D_OUT), jnp.float32),
                   jax.ShapeDtypeStruct((L, NN, D_EDGE_OUT), jnp.float32)],
        input_output_aliases=aliases,
    )(x, evf, nodetrg, *extra, *flat)
    return out, ev2


_NW = 32
_NH = 2
_LH = L // _NH
_RPW = _LH // _NW
_IPW = _RPW * NN
_ABATCH = 8
_GCHUNK = 128
_WPR = L // 4


def _sc_body(adj_hbm, x_hbm, out_hbm, adjbuf0, adjbuf1, idxrow, idxall,
             rows0, rows1, sa0, sa1, sg0, sg1, so0, so1):
    wid = lax.axis_index("s") * 2 + lax.axis_index("c")
    row0 = wid * _RPW
    adjb = [adjbuf0, adjbuf1]
    rows = [rows0, rows1]
    sa, sg, so = [sa0, sa1], [sg0, sg1], [so0, so1]
    nb = _RPW // _ABATCH

    def start_adj(b):
        return pltpu.async_copy(
            adj_hbm.at[pl.ds((row0 + b * _ABATCH) * L, _ABATCH * L)],
            adjb[b % 2], sa[b % 2])

    def start_gather(b):
        return pltpu.async_copy(
            x_hbm.at[idxall.at[pl.ds(b * _GCHUNK, _GCHUNK)]],
            rows[b % 2], sg[b % 2])

    def start_out(b):
        return pltpu.async_copy(
            rows[b % 2],
            out_hbm.at[pl.ds(wid * _IPW + b * _GCHUNK, _GCHUNK)], so[b % 2])

    def scan_batch(buf, b):
        def scan_row(r, _):
            rbase = r * L

            def chunk(carry, want_true):
                k, ptr = carry
                for u in range(2):
                    vals = buf[pl.ds(rbase + (k + u) * NN, NN)]
                    mask = (vals != 0) if want_true else (vals == 0)
                    inds = lax.iota(jnp.int32, NN) + (k + u) * NN
                    plsc.store_compressed(idxrow.at[pl.ds(ptr, NN)], inds,
                                          mask=mask)
                    ptr = ptr + jnp.sum(mask.astype(jnp.int32))
                return k + 2, ptr

            def cont(carry):
                k, ptr = carry
                return jnp.logical_and(ptr < NN, k < L // NN)

            _, ptr = lax.while_loop(cont, lambda c: chunk(c, True), (0, 0))
            _, _ = lax.while_loop(cont, lambda c: chunk(c, False), (0, ptr))
            row = b * _ABATCH + r
            idxall[pl.ds(row * NN, NN)] = idxrow[pl.ds(0, NN)]
            return 0

        lax.fori_loop(0, _ABATCH, scan_row, 0)

    h_adj, h_g, h_o = {}, {}, {}
    h_adj[0] = start_adj(0)
    for b in range(nb):
        if b + 1 < nb:
            h_adj[b + 1] = start_adj(b + 1)
        h_adj[b].wait()
        scan_batch(adjb[b % 2], b)
        if b >= 1:
            h_g[b - 1].wait()
            h_o[b - 1] = start_out(b - 1)
        if b >= 2:
            h_o[b - 2].wait()
        h_g[b] = start_gather(b)
    h_g[nb - 1].wait()
    h_o[nb - 1] = start_out(nb - 1)
    h_o[nb - 2].wait()
    h_o[nb - 1].wait()


def _neighbors(x, adj_i32):
    mesh = plsc.VectorSubcoreMesh(core_axis_name="c", subcore_axis_name="s")
    f = functools.partial(
        pl.kernel,
        mesh=mesh,
        compiler_params=pltpu.CompilerParams(needs_layout_passes=False),
        out_type=jax.ShapeDtypeStruct((_LH * NN, D_IN), jnp.float32),
        scratch_types=[
            pltpu.VMEM((_ABATCH * L,), jnp.int32),
            pltpu.VMEM((_ABATCH * L,), jnp.int32),
            pltpu.VMEM((3 * NN,), jnp.int32),
            pltpu.VMEM((_IPW,), jnp.int32),
            pltpu.VMEM((_GCHUNK, D_IN), jnp.float32),
            pltpu.VMEM((_GCHUNK, D_IN), jnp.float32),
            pltpu.SemaphoreType.DMA,
            pltpu.SemaphoreType.DMA,
            pltpu.SemaphoreType.DMA,
            pltpu.SemaphoreType.DMA,
            pltpu.SemaphoreType.DMA,
            pltpu.SemaphoreType.DMA,
        ],
    )(_sc_body)
    return f(adj_i32, x)


@jax.jit
def kernel(x, edgevec, adjmat, params):
    evf = edgevec.reshape(L * NN, D_EDGE_IN)
    flat = _flatten_params(params)
    prev = None
    for h in range(_NH):
        adj_h = adjmat[h * _LH:(h + 1) * _LH].reshape(_LH * L).astype(jnp.int32)
        ntg_h = _neighbors(x, adj_h)
        prev = _mlp(x, evf, ntg_h, flat, h, prev)
    out, ev2 = prev
    return out, ev2
